# Initial kernel scaffold; baseline (speedup 1.0000x reference)
import jax
import jax.numpy as jnp
from jax.experimental import pallas as pl


def kernel(population_id, allele_freq_features, table, W, b, gamma, beta):
    raise NotImplementedError("write your pallas kernel here")


# speedup vs baseline: 1.1913x; 1.1913x over previous
import jax
import jax.numpy as jnp
from jax import lax
from jax.experimental import pallas as pl
from jax.experimental.pallas import tpu as pltpu
from jax.experimental.pallas import tpu_sc as plsc

_N_POP = 1000
_EMBED_DIM = 32
_N_AF = 16
_TOTAL_DIM = _EMBED_DIM + _N_AF
_BATCH = 16384

_NC = 2
_NS = 16
_NW = _NC * _NS
_BPW = _BATCH // _NW


def _sc_gather_body(table_hbm, idx_hbm, out_hbm, idx_v, rows_v, sem):
    wid = lax.axis_index("s") * _NC + lax.axis_index("c")
    base = wid * _BPW
    pltpu.sync_copy(idx_hbm.at[pl.ds(base, _BPW)], idx_v)
    pltpu.async_copy(table_hbm.at[idx_v], rows_v, sem).wait()
    pltpu.sync_copy(rows_v, out_hbm.at[pl.ds(base, _BPW)])


def _sc_gather(table, idx):
    mesh = plsc.VectorSubcoreMesh(core_axis_name="c", subcore_axis_name="s")
    return pl.kernel(
        _sc_gather_body,
        out_type=jax.ShapeDtypeStruct((_BATCH, _EMBED_DIM), jnp.float32),
        mesh=mesh,
        compiler_params=pltpu.CompilerParams(use_tc_tiling_on_sc=False),
        scratch_types=[
            pltpu.VMEM((_BPW,), jnp.int32),
            pltpu.VMEM((_BPW, _EMBED_DIM), jnp.float32),
            pltpu.SemaphoreType.DMA,
        ],
    )(table, idx)


_BLK = 2048


def _dense_body(emb_ref, af_ref, w1_ref, w2_ref, b_ref, g_ref, beta_ref, out_ref):
    h = lax.dot_general(
        emb_ref[:], w1_ref[:],
        (((1,), (1,)), ((), ())),
        preferred_element_type=jnp.float32,
    )
    h = h + lax.dot_general(
        af_ref[:], w2_ref[:],
        (((1,), (1,)), ((), ())),
        preferred_element_type=jnp.float32,
    )
    h = h + b_ref[:]
    mu = jnp.mean(h, axis=1, keepdims=True)
    xc = h - mu
    var = jnp.mean(xc * xc, axis=1, keepdims=True)
    y = xc * lax.rsqrt(var + 1e-5) * g_ref[:] + beta_ref[:]
    out_ref[:] = jnp.maximum(y, 0.0)


def _dense(emb, af, W, b, gamma, beta):
    w1 = W[:, :_EMBED_DIM]
    w2 = W[:, _EMBED_DIM:]
    b2 = b.reshape(1, _TOTAL_DIM)
    g2 = gamma.reshape(1, _TOTAL_DIM)
    beta2 = beta.reshape(1, _TOTAL_DIM)
    grid = (_BATCH // _BLK,)
    return pl.pallas_call(
        _dense_body,
        grid=grid,
        in_specs=[
            pl.BlockSpec((_BLK, _EMBED_DIM), lambda i: (i, 0)),
            pl.BlockSpec((_BLK, _N_AF), lambda i: (i, 0)),
            pl.BlockSpec((_TOTAL_DIM, _EMBED_DIM), lambda i: (0, 0)),
            pl.BlockSpec((_TOTAL_DIM, _N_AF), lambda i: (0, 0)),
            pl.BlockSpec((1, _TOTAL_DIM), lambda i: (0, 0)),
            pl.BlockSpec((1, _TOTAL_DIM), lambda i: (0, 0)),
            pl.BlockSpec((1, _TOTAL_DIM), lambda i: (0, 0)),
        ],
        out_specs=pl.BlockSpec((_BLK, _TOTAL_DIM), lambda i: (i, 0)),
        out_shape=jax.ShapeDtypeStruct((_BATCH, _TOTAL_DIM), jnp.float32),
    )(emb, af, w1, w2, b2, g2, beta2)


def kernel(population_id, allele_freq_features, table, W, b, gamma, beta):
    emb = _sc_gather(table, population_id)
    return _dense(emb, allele_freq_features, W, b, gamma, beta)


# speedup vs baseline: 2.1050x; 1.7669x over previous
import jax
import jax.numpy as jnp
from jax import lax
from jax.experimental import pallas as pl
from jax.experimental.pallas import tpu as pltpu
from jax.experimental.pallas import tpu_sc as plsc

_N_POP = 1000
_EMBED_DIM = 32
_N_AF = 16
_TOTAL_DIM = _EMBED_DIM + _N_AF
_BATCH = 16384

_NC = 2
_NS = 16
_NW = _NC * _NS
_BPW = _BATCH // _NW
_PACK = 128 // _EMBED_DIM
_BLK = 4096
_SUB = _BLK // _PACK


def _sc_gather_body(table_hbm, idx_hbm, out_hbm, idx_v, rows_v, sem):
    wid = lax.axis_index("s") * _NC + lax.axis_index("c")
    base = wid * _BPW
    pltpu.sync_copy(idx_hbm.at[pl.ds(base, _BPW)], idx_v)
    pltpu.async_copy(table_hbm.at[idx_v], rows_v, sem).wait()
    sub = wid % (_NW // (_BATCH // _BLK))
    blk = wid // (_NW // (_BATCH // _BLK))
    r0 = _SUB * blk + _BPW * (sub % 2)
    c0 = _EMBED_DIM * (sub // 2)
    pltpu.sync_copy(rows_v, out_hbm.at[pl.ds(r0, _BPW), pl.ds(c0, _EMBED_DIM)])


def _sc_gather_packed(table, idx):
    mesh = plsc.VectorSubcoreMesh(core_axis_name="c", subcore_axis_name="s")
    return pl.kernel(
        _sc_gather_body,
        out_type=jax.ShapeDtypeStruct((_BATCH // _PACK, 128), jnp.float32),
        mesh=mesh,
        compiler_params=pltpu.CompilerParams(use_tc_tiling_on_sc=False),
        scratch_types=[
            pltpu.VMEM((_BPW,), jnp.int32),
            pltpu.VMEM((_BPW, _EMBED_DIM), jnp.float32),
            pltpu.SemaphoreType.DMA,
        ],
    )(table, idx)


def _dense_body(e4_ref, at_ref, w1_ref, w2_ref, b_ref, g_ref, beta_ref, out_ref):
    e4 = e4_ref[:]
    parts = [
        lax.dot_general(
            w1_ref[:], e4[:, q * _EMBED_DIM:(q + 1) * _EMBED_DIM],
            (((1,), (1,)), ((), ())),
            preferred_element_type=jnp.float32,
        )
        for q in range(_PACK)
    ]
    h = jnp.concatenate(parts, axis=1)
    h = h + lax.dot_general(
        w2_ref[:], at_ref[:],
        (((1,), (0,)), ((), ())),
        preferred_element_type=jnp.float32,
    )
    h = h + b_ref[:]
    mu = jnp.mean(h, axis=0, keepdims=True)
    xc = h - mu
    var = jnp.mean(xc * xc, axis=0, keepdims=True)
    y = xc * lax.rsqrt(var + 1e-5) * g_ref[:] + beta_ref[:]
    out_ref[:] = jnp.maximum(y, 0.0)


def _dense_t(emb4, afT, W, b, gamma, beta):
    w1 = W[:, :_EMBED_DIM]
    w2 = W[:, _EMBED_DIM:]
    b2 = b.reshape(_TOTAL_DIM, 1)
    g2 = gamma.reshape(_TOTAL_DIM, 1)
    beta2 = beta.reshape(_TOTAL_DIM, 1)
    grid = (_BATCH // _BLK,)
    return pl.pallas_call(
        _dense_body,
        grid=grid,
        in_specs=[
            pl.BlockSpec((_SUB, 128), lambda i: (i, 0)),
            pl.BlockSpec((_N_AF, _BLK), lambda i: (0, i)),
            pl.BlockSpec((_TOTAL_DIM, _EMBED_DIM), lambda i: (0, 0)),
            pl.BlockSpec((_TOTAL_DIM, _N_AF), lambda i: (0, 0)),
            pl.BlockSpec((_TOTAL_DIM, 1), lambda i: (0, 0)),
            pl.BlockSpec((_TOTAL_DIM, 1), lambda i: (0, 0)),
            pl.BlockSpec((_TOTAL_DIM, 1), lambda i: (0, 0)),
        ],
        out_specs=pl.BlockSpec((_TOTAL_DIM, _BLK), lambda i: (0, i)),
        out_shape=jax.ShapeDtypeStruct((_TOTAL_DIM, _BATCH), jnp.float32),
    )(emb4, afT, w1, w2, b2, g2, beta2)


def kernel(population_id, allele_freq_features, table, W, b, gamma, beta):
    emb4 = _sc_gather_packed(table, population_id)
    outT = _dense_t(emb4, allele_freq_features.T, W, b, gamma, beta)
    return outT.T
